# baseline (speedup 1.0000x reference)
import functools

import jax
import jax.numpy as jnp
from jax import lax
from jax.experimental import pallas as pl
from jax.experimental.pallas import tpu as pltpu
from jax.experimental.pallas import tpu_sc as plsc

NUM_CORES = 2
NUM_SUBCORES = 16
NUM_WORKERS = NUM_CORES * NUM_SUBCORES
LANES = 16

BATCH = 16384
DIM = 32
ROWS_PER_LINE = 128 // DIM
TAB_ROWS = 1000000 // ROWS_PER_LINE
ROWS_PER_WORKER = BATCH // NUM_WORKERS
CHUNK = 128
NCHUNKS = ROWS_PER_WORKER // CHUNK
GROUPS = CHUNK // LANES


def _gmf_body(u_rm, i_rm, w_hbm, b_hbm, uidx_hbm, iidx_hbm, out_hbm,
              uidx_f, iidx_f, urid, irid, ubuf, ibuf, w_v, b_v, out_v, sem):
    wid = lax.axis_index("s") * NUM_CORES + lax.axis_index("c")
    base = wid * ROWS_PER_WORKER

    pltpu.sync_copy(uidx_hbm.at[pl.ds(base, ROWS_PER_WORKER)], uidx_f)
    pltpu.sync_copy(iidx_hbm.at[pl.ds(base, ROWS_PER_WORKER)], iidx_f)
    pltpu.sync_copy(w_hbm, w_v)
    pltpu.sync_copy(b_hbm, b_v)

    for j in range(ROWS_PER_WORKER // LANES):
        dst = (j * LANES) // CHUNK, pl.ds((j * LANES) % CHUNK, LANES)
        urid[dst] = uidx_f[pl.ds(j * LANES, LANES)] // ROWS_PER_LINE
        irid[dst] = iidx_f[pl.ds(j * LANES, LANES)] // ROWS_PER_LINE

    iota = lax.iota(jnp.int32, LANES)
    bias = b_v[...]
    w_lo = w_v[pl.ds(0, LANES)]
    w_hi = w_v[pl.ds(LANES, LANES)]

    for c in range(NCHUNKS):
        cu = pltpu.async_copy(u_rm.at[urid.at[c]], ubuf, sem)
        ci = pltpu.async_copy(i_rm.at[irid.at[c]], ibuf, sem)
        cu.wait()
        ci.wait()

        def group(g, carry, c=c):
            off = c * CHUNK + g * LANES
            uix = uidx_f[pl.ds(off, LANES)]
            iix = iidx_f[pl.ds(off, LANES)]
            rid = g * LANES + iota
            ul0 = (uix % ROWS_PER_LINE) * DIM
            il0 = (iix % ROWS_PER_LINE) * DIM
            acc = bias
            for d in range(DIM):
                ug = plsc.load_gather(ubuf, [rid, ul0 + d])
                vg = plsc.load_gather(ibuf, [rid, il0 + d])
                w_s = (w_lo if d < LANES else w_hi)[d % LANES]
                acc = acc + ug * vg * w_s
            out_v[pl.ds(off, LANES)] = 1.0 / (1.0 + jnp.exp(-acc))
            return carry

        lax.fori_loop(0, GROUPS, group, 0)

    pltpu.sync_copy(out_v, out_hbm.at[pl.ds(base, ROWS_PER_WORKER)])


@functools.partial(jax.jit, static_argnames=())
def _gmf(u_rm, i_rm, w_flat, b_vec, uidx, iidx):
    mesh = plsc.VectorSubcoreMesh(core_axis_name="c", subcore_axis_name="s")
    run = pl.kernel(
        _gmf_body,
        out_type=jax.ShapeDtypeStruct((BATCH,), jnp.float32),
        mesh=mesh,
        scratch_types=[
            pltpu.VMEM((ROWS_PER_WORKER,), jnp.int32),
            pltpu.VMEM((ROWS_PER_WORKER,), jnp.int32),
            pltpu.VMEM((NCHUNKS, CHUNK), jnp.int32),
            pltpu.VMEM((NCHUNKS, CHUNK), jnp.int32),
            pltpu.VMEM((CHUNK, 128), jnp.float32),
            pltpu.VMEM((CHUNK, 128), jnp.float32),
            pltpu.VMEM((DIM,), jnp.float32),
            pltpu.VMEM((LANES,), jnp.float32),
            pltpu.VMEM((ROWS_PER_WORKER,), jnp.float32),
            pltpu.SemaphoreType.DMA,
        ],
        compiler_params=pltpu.CompilerParams(
            needs_layout_passes=False, use_tc_tiling_on_sc=True),
    )
    return run(u_rm, i_rm, w_flat, b_vec, uidx, iidx)


def kernel(user_table, item_table, fc_w, fc_b, user_indices, item_indices):
    u_rm = user_table.reshape(TAB_ROWS, 128)
    i_rm = item_table.reshape(TAB_ROWS, 128)
    w_flat = fc_w.reshape(DIM).astype(jnp.float32)
    b_vec = jnp.broadcast_to(fc_b.astype(jnp.float32), (LANES,))
    uidx = user_indices.astype(jnp.int32)
    iidx = item_indices.astype(jnp.int32)
    out = _gmf(u_rm, i_rm, w_flat, b_vec, uidx, iidx)
    return out.reshape(BATCH, 1)


# speedup vs baseline: 1.1277x; 1.1277x over previous
import functools

import jax
import jax.numpy as jnp
from jax import lax
from jax.experimental import pallas as pl
from jax.experimental.pallas import tpu as pltpu
from jax.experimental.pallas import tpu_sc as plsc

NUM_CORES = 2
NUM_SUBCORES = 16
NUM_WORKERS = NUM_CORES * NUM_SUBCORES
LANES = 16

BATCH = 16384
DIM = 32
NUM_ROWS = 1000000
ROWS_PER_LINE = 128 // DIM
TAB_ROWS = NUM_ROWS // ROWS_PER_LINE
ROWS_PER_WORKER = BATCH // NUM_WORKERS
CHUNK = 128
NCHUNKS = ROWS_PER_WORKER // CHUNK
GROUPS = CHUNK // LANES

BLK_R = 512
BLK_U = BLK_R * ROWS_PER_LINE
RELAYOUT_GRID = (TAB_ROWS + BLK_R - 1) // BLK_R


def _relayout_body(u_ref, i_ref, uo_ref, io_ref):
    for x_ref, o_ref in ((u_ref, uo_ref), (i_ref, io_ref)):
        y = jnp.swapaxes(x_ref[...], 0, 1)
        y3 = jnp.reshape(y, (BLK_R, ROWS_PER_LINE, DIM))
        o_ref[...] = jnp.concatenate(
            [y3[:, j, :] for j in range(ROWS_PER_LINE)], axis=1)


def _relayout(ut_t, it_t):
    spec_in = pl.BlockSpec((DIM, BLK_U), lambda i: (0, i))
    spec_out = pl.BlockSpec((BLK_R, 128), lambda i: (i, 0))
    return pl.pallas_call(
        _relayout_body,
        grid=(RELAYOUT_GRID,),
        in_specs=[spec_in, spec_in],
        out_specs=[spec_out, spec_out],
        out_shape=[jax.ShapeDtypeStruct((TAB_ROWS, 128), jnp.float32)] * 2,
    )(ut_t, it_t)


def _gmf_body(u_rm, i_rm, w_hbm, b_hbm, uidx_hbm, iidx_hbm, out_hbm,
              uidx_f, iidx_f, urid, irid, ubuf, ibuf, w_v, b_v, out_v, sem):
    wid = lax.axis_index("s") * NUM_CORES + lax.axis_index("c")
    base = wid * ROWS_PER_WORKER

    pltpu.sync_copy(uidx_hbm.at[pl.ds(base, ROWS_PER_WORKER)], uidx_f)
    pltpu.sync_copy(iidx_hbm.at[pl.ds(base, ROWS_PER_WORKER)], iidx_f)
    pltpu.sync_copy(w_hbm, w_v)
    pltpu.sync_copy(b_hbm, b_v)

    for j in range(ROWS_PER_WORKER // LANES):
        dst = (j * LANES) // CHUNK, pl.ds((j * LANES) % CHUNK, LANES)
        urid[dst] = uidx_f[pl.ds(j * LANES, LANES)] // ROWS_PER_LINE
        irid[dst] = iidx_f[pl.ds(j * LANES, LANES)] // ROWS_PER_LINE

    iota = lax.iota(jnp.int32, LANES)
    bias = b_v[...]
    w_lo = w_v[pl.ds(0, LANES)]
    w_hi = w_v[pl.ds(LANES, LANES)]

    for c in range(NCHUNKS):
        cu = pltpu.async_copy(u_rm.at[urid.at[c]], ubuf, sem)
        ci = pltpu.async_copy(i_rm.at[irid.at[c]], ibuf, sem)
        cu.wait()
        ci.wait()

        def group(g, carry, c=c):
            off = c * CHUNK + g * LANES
            uix = uidx_f[pl.ds(off, LANES)]
            iix = iidx_f[pl.ds(off, LANES)]
            rid = g * LANES + iota
            ul0 = (uix % ROWS_PER_LINE) * DIM
            il0 = (iix % ROWS_PER_LINE) * DIM
            acc = bias
            for d in range(DIM):
                ug = plsc.load_gather(ubuf, [rid, ul0 + d])
                vg = plsc.load_gather(ibuf, [rid, il0 + d])
                w_s = (w_lo if d < LANES else w_hi)[d % LANES]
                acc = acc + ug * vg * w_s
            out_v[pl.ds(off, LANES)] = 1.0 / (1.0 + jnp.exp(-acc))
            return carry

        lax.fori_loop(0, GROUPS, group, 0)

    pltpu.sync_copy(out_v, out_hbm.at[pl.ds(base, ROWS_PER_WORKER)])


@functools.partial(jax.jit, static_argnames=())
def _gmf(u_rm, i_rm, w_flat, b_vec, uidx, iidx):
    mesh = plsc.VectorSubcoreMesh(core_axis_name="c", subcore_axis_name="s")
    run = pl.kernel(
        _gmf_body,
        out_type=jax.ShapeDtypeStruct((BATCH,), jnp.float32),
        mesh=mesh,
        scratch_types=[
            pltpu.VMEM((ROWS_PER_WORKER,), jnp.int32),
            pltpu.VMEM((ROWS_PER_WORKER,), jnp.int32),
            pltpu.VMEM((NCHUNKS, CHUNK), jnp.int32),
            pltpu.VMEM((NCHUNKS, CHUNK), jnp.int32),
            pltpu.VMEM((CHUNK, 128), jnp.float32),
            pltpu.VMEM((CHUNK, 128), jnp.float32),
            pltpu.VMEM((DIM,), jnp.float32),
            pltpu.VMEM((LANES,), jnp.float32),
            pltpu.VMEM((ROWS_PER_WORKER,), jnp.float32),
            pltpu.SemaphoreType.DMA,
        ],
        compiler_params=pltpu.CompilerParams(
            needs_layout_passes=False, use_tc_tiling_on_sc=True),
    )
    return run(u_rm, i_rm, w_flat, b_vec, uidx, iidx)


def kernel(user_table, item_table, fc_w, fc_b, user_indices, item_indices):
    u_rm, i_rm = _relayout(user_table.T, item_table.T)
    w_flat = fc_w.reshape(DIM).astype(jnp.float32)
    b_vec = jnp.broadcast_to(fc_b.astype(jnp.float32), (LANES,))
    uidx = user_indices.astype(jnp.int32)
    iidx = item_indices.astype(jnp.int32)
    out = _gmf(u_rm, i_rm, w_flat, b_vec, uidx, iidx)
    return out.reshape(BATCH, 1)


# speedup vs baseline: 2.1736x; 1.9276x over previous
import functools

import jax
import jax.numpy as jnp
from jax import lax
from jax.experimental import pallas as pl
from jax.experimental.pallas import tpu as pltpu
from jax.experimental.pallas import tpu_sc as plsc

NUM_CORES = 2
NUM_SUBCORES = 16
NUM_WORKERS = NUM_CORES * NUM_SUBCORES
LANES = 16

BATCH = 16384
DIM = 32
NUM_ROWS = 1000000
ROWS_PER_LINE = 128 // DIM
TAB_ROWS = NUM_ROWS // ROWS_PER_LINE
ROWS_PER_WORKER = BATCH // NUM_WORKERS
CHUNK = 128
NCHUNKS = ROWS_PER_WORKER // CHUNK
GROUPS = CHUNK // LANES

BLK_R = 512
BLK_U = BLK_R * ROWS_PER_LINE
RELAYOUT_GRID = (TAB_ROWS + BLK_R - 1) // BLK_R
OUT_ROWS = RELAYOUT_GRID * BLK_R


def _relayout_body(u_ref, i_ref, uo_ref, io_ref):
    for x_ref, o_ref in ((u_ref, uo_ref), (i_ref, io_ref)):
        x = x_ref[...]
        xs = jnp.concatenate(
            [x[:, c * BLK_R:(c + 1) * BLK_R] for c in range(ROWS_PER_LINE)],
            axis=0)
        o_ref[...] = jnp.swapaxes(xs, 0, 1)


def _relayout(ut_t, it_t):
    spec_in = pl.BlockSpec((DIM, BLK_U), lambda i: (0, i))
    spec_out = pl.BlockSpec((BLK_R, 128), lambda i: (i, 0))
    return pl.pallas_call(
        _relayout_body,
        grid=(RELAYOUT_GRID,),
        in_specs=[spec_in, spec_in],
        out_specs=[spec_out, spec_out],
        out_shape=[jax.ShapeDtypeStruct((OUT_ROWS, 128), jnp.float32)] * 2,
    )(ut_t, it_t)


def _gmf_body(u_rm, i_rm, w_hbm, b_hbm, uidx_hbm, iidx_hbm, out_hbm,
              uidx_f, iidx_f, urid, irid, ubuf, ibuf, w_v, b_v, out_v, sem):
    wid = lax.axis_index("s") * NUM_CORES + lax.axis_index("c")
    base = wid * ROWS_PER_WORKER

    pltpu.sync_copy(uidx_hbm.at[pl.ds(base, ROWS_PER_WORKER)], uidx_f)
    pltpu.sync_copy(iidx_hbm.at[pl.ds(base, ROWS_PER_WORKER)], iidx_f)
    pltpu.sync_copy(w_hbm, w_v)
    pltpu.sync_copy(b_hbm, b_v)

    for j in range(ROWS_PER_WORKER // LANES):
        dst = (j * LANES) // CHUNK, pl.ds((j * LANES) % CHUNK, LANES)
        uv = uidx_f[pl.ds(j * LANES, LANES)]
        iv = iidx_f[pl.ds(j * LANES, LANES)]
        urid[dst] = ((uv >> 11) << 9) + (uv & 511)
        irid[dst] = ((iv >> 11) << 9) + (iv & 511)

    iota = lax.iota(jnp.int32, LANES)
    bias = b_v[...]
    w_lo = w_v[pl.ds(0, LANES)]
    w_hi = w_v[pl.ds(LANES, LANES)]

    for c in range(NCHUNKS):
        cu = pltpu.async_copy(u_rm.at[urid.at[c]], ubuf, sem)
        ci = pltpu.async_copy(i_rm.at[irid.at[c]], ibuf, sem)
        cu.wait()
        ci.wait()

        def group(g, carry, c=c):
            off = c * CHUNK + g * LANES
            uix = uidx_f[pl.ds(off, LANES)]
            iix = iidx_f[pl.ds(off, LANES)]
            rid = g * LANES + iota
            ul0 = ((uix >> 9) & 3) * DIM
            il0 = ((iix >> 9) & 3) * DIM
            acc = bias
            for d in range(DIM):
                ug = plsc.load_gather(ubuf, [rid, ul0 + d])
                vg = plsc.load_gather(ibuf, [rid, il0 + d])
                w_s = (w_lo if d < LANES else w_hi)[d % LANES]
                acc = acc + ug * vg * w_s
            out_v[pl.ds(off, LANES)] = 1.0 / (1.0 + jnp.exp(-acc))
            return carry

        lax.fori_loop(0, GROUPS, group, 0)

    pltpu.sync_copy(out_v, out_hbm.at[pl.ds(base, ROWS_PER_WORKER)])


@functools.partial(jax.jit, static_argnames=())
def _gmf(u_rm, i_rm, w_flat, b_vec, uidx, iidx):
    mesh = plsc.VectorSubcoreMesh(core_axis_name="c", subcore_axis_name="s")
    run = pl.kernel(
        _gmf_body,
        out_type=jax.ShapeDtypeStruct((BATCH,), jnp.float32),
        mesh=mesh,
        scratch_types=[
            pltpu.VMEM((ROWS_PER_WORKER,), jnp.int32),
            pltpu.VMEM((ROWS_PER_WORKER,), jnp.int32),
            pltpu.VMEM((NCHUNKS, CHUNK), jnp.int32),
            pltpu.VMEM((NCHUNKS, CHUNK), jnp.int32),
            pltpu.VMEM((CHUNK, 128), jnp.float32),
            pltpu.VMEM((CHUNK, 128), jnp.float32),
            pltpu.VMEM((DIM,), jnp.float32),
            pltpu.VMEM((LANES,), jnp.float32),
            pltpu.VMEM((ROWS_PER_WORKER,), jnp.float32),
            pltpu.SemaphoreType.DMA,
        ],
        compiler_params=pltpu.CompilerParams(
            needs_layout_passes=False, use_tc_tiling_on_sc=True),
    )
    return run(u_rm, i_rm, w_flat, b_vec, uidx, iidx)


def kernel(user_table, item_table, fc_w, fc_b, user_indices, item_indices):
    u_rm, i_rm = _relayout(user_table.T, item_table.T)
    w_flat = fc_w.reshape(DIM).astype(jnp.float32)
    b_vec = jnp.broadcast_to(fc_b.astype(jnp.float32), (LANES,))
    uidx = user_indices.astype(jnp.int32)
    iidx = item_indices.astype(jnp.int32)
    out = _gmf(u_rm, i_rm, w_flat, b_vec, uidx, iidx)
    return out.reshape(BATCH, 1)


# speedup vs baseline: 3.8058x; 1.7509x over previous
import functools

import jax
import jax.numpy as jnp
from jax import lax
from jax.experimental import pallas as pl
from jax.experimental.pallas import tpu as pltpu
from jax.experimental.pallas import tpu_sc as plsc

NUM_CORES = 2
NUM_SUBCORES = 16
NUM_WORKERS = NUM_CORES * NUM_SUBCORES
LANES = 16

BATCH = 16384
DIM = 32
NUM_ROWS = 1000000
ROWS_PER_LINE = 128 // DIM
TAB_ROWS = NUM_ROWS // ROWS_PER_LINE
ROWS_PER_WORKER = BATCH // NUM_WORKERS
CHUNK = 128
NCHUNKS = ROWS_PER_WORKER // CHUNK
GROUPS = CHUNK // LANES

BLK_R = 2048
BLK_U = BLK_R * ROWS_PER_LINE
RELAYOUT_GRID = (TAB_ROWS + BLK_R - 1) // BLK_R
OUT_ROWS = RELAYOUT_GRID * BLK_R
BLK_SHIFT = BLK_R.bit_length() - 1
BLK_MASK = BLK_R - 1


def _relayout_body(u_ref, i_ref, uo_ref, io_ref):
    for x_ref, o_ref in ((u_ref, uo_ref), (i_ref, io_ref)):
        x = x_ref[...]
        xs = jnp.concatenate(
            [x[:, c * BLK_R:(c + 1) * BLK_R] for c in range(ROWS_PER_LINE)],
            axis=0)
        o_ref[...] = jnp.swapaxes(xs, 0, 1)


def _relayout(ut_t, it_t):
    spec_in = pl.BlockSpec((DIM, BLK_U), lambda i: (0, i))
    spec_out = pl.BlockSpec((BLK_R, 128), lambda i: (i, 0))
    return pl.pallas_call(
        _relayout_body,
        grid=(RELAYOUT_GRID,),
        in_specs=[spec_in, spec_in],
        out_specs=[spec_out, spec_out],
        out_shape=[jax.ShapeDtypeStruct((OUT_ROWS, 128), jnp.float32)] * 2,
    )(ut_t, it_t)


def _gmf_body(u_rm, i_rm, w_hbm, b_hbm, uidx_hbm, iidx_hbm, out_hbm,
              uidx_f, iidx_f, urid, irid, ubuf, ibuf, w_v, b_v, out_v, sem):
    wid = lax.axis_index("s") * NUM_CORES + lax.axis_index("c")
    base = wid * ROWS_PER_WORKER

    pltpu.sync_copy(uidx_hbm.at[pl.ds(base, ROWS_PER_WORKER)], uidx_f)
    pltpu.sync_copy(iidx_hbm.at[pl.ds(base, ROWS_PER_WORKER)], iidx_f)
    pltpu.sync_copy(w_hbm, w_v)
    pltpu.sync_copy(b_hbm, b_v)

    for j in range(ROWS_PER_WORKER // LANES):
        dst = (j * LANES) // CHUNK, pl.ds((j * LANES) % CHUNK, LANES)
        uv = uidx_f[pl.ds(j * LANES, LANES)]
        iv = iidx_f[pl.ds(j * LANES, LANES)]
        urid[dst] = ((uv >> (BLK_SHIFT + 2)) << BLK_SHIFT) + (uv & BLK_MASK)
        irid[dst] = ((iv >> (BLK_SHIFT + 2)) << BLK_SHIFT) + (iv & BLK_MASK)

    iota = lax.iota(jnp.int32, LANES)
    bias = b_v[...]
    w_lo = w_v[pl.ds(0, LANES)]
    w_hi = w_v[pl.ds(LANES, LANES)]

    for c in range(NCHUNKS):
        cu = pltpu.async_copy(u_rm.at[urid.at[c]], ubuf, sem)
        ci = pltpu.async_copy(i_rm.at[irid.at[c]], ibuf, sem)
        cu.wait()
        ci.wait()

        def group(g, carry, c=c):
            off = c * CHUNK + g * LANES
            uix = uidx_f[pl.ds(off, LANES)]
            iix = iidx_f[pl.ds(off, LANES)]
            rid = g * LANES + iota
            ul0 = ((uix >> BLK_SHIFT) & 3) * DIM
            il0 = ((iix >> BLK_SHIFT) & 3) * DIM
            acc = bias
            for d in range(DIM):
                ug = plsc.load_gather(ubuf, [rid, ul0 + d])
                vg = plsc.load_gather(ibuf, [rid, il0 + d])
                w_s = (w_lo if d < LANES else w_hi)[d % LANES]
                acc = acc + ug * vg * w_s
            out_v[pl.ds(off, LANES)] = 1.0 / (1.0 + jnp.exp(-acc))
            return carry

        lax.fori_loop(0, GROUPS, group, 0)

    pltpu.sync_copy(out_v, out_hbm.at[pl.ds(base, ROWS_PER_WORKER)])


@functools.partial(jax.jit, static_argnames=())
def _gmf(u_rm, i_rm, w_flat, b_vec, uidx, iidx):
    mesh = plsc.VectorSubcoreMesh(core_axis_name="c", subcore_axis_name="s")
    run = pl.kernel(
        _gmf_body,
        out_type=jax.ShapeDtypeStruct((BATCH,), jnp.float32),
        mesh=mesh,
        scratch_types=[
            pltpu.VMEM((ROWS_PER_WORKER,), jnp.int32),
            pltpu.VMEM((ROWS_PER_WORKER,), jnp.int32),
            pltpu.VMEM((NCHUNKS, CHUNK), jnp.int32),
            pltpu.VMEM((NCHUNKS, CHUNK), jnp.int32),
            pltpu.VMEM((CHUNK, 128), jnp.float32),
            pltpu.VMEM((CHUNK, 128), jnp.float32),
            pltpu.VMEM((DIM,), jnp.float32),
            pltpu.VMEM((LANES,), jnp.float32),
            pltpu.VMEM((ROWS_PER_WORKER,), jnp.float32),
            pltpu.SemaphoreType.DMA,
        ],
        compiler_params=pltpu.CompilerParams(
            needs_layout_passes=False, use_tc_tiling_on_sc=True),
    )
    return run(u_rm, i_rm, w_flat, b_vec, uidx, iidx)


def kernel(user_table, item_table, fc_w, fc_b, user_indices, item_indices):
    u_rm, i_rm = _relayout(user_table.T, item_table.T)
    w_flat = fc_w.reshape(DIM).astype(jnp.float32)
    b_vec = jnp.broadcast_to(fc_b.astype(jnp.float32), (LANES,))
    uidx = user_indices.astype(jnp.int32)
    iidx = item_indices.astype(jnp.int32)
    out = _gmf(u_rm, i_rm, w_flat, b_vec, uidx, iidx)
    return out.reshape(BATCH, 1)


# speedup vs baseline: 4.4413x; 1.1670x over previous
import functools

import jax
import jax.numpy as jnp
from jax import lax
from jax.experimental import pallas as pl
from jax.experimental.pallas import tpu as pltpu
from jax.experimental.pallas import tpu_sc as plsc

NUM_CORES = 2
NUM_SUBCORES = 16
NUM_WORKERS = NUM_CORES * NUM_SUBCORES
LANES = 16

BATCH = 16384
DIM = 32
NUM_ROWS = 1000000
ROWS_PER_LINE = 128 // DIM
TAB_ROWS = NUM_ROWS // ROWS_PER_LINE
ROWS_PER_WORKER = BATCH // NUM_WORKERS
CHUNK = 128
NCHUNKS = ROWS_PER_WORKER // CHUNK
GROUPS = CHUNK // LANES

BLK_R = 8192
BLK_U = BLK_R * ROWS_PER_LINE
RELAYOUT_GRID = (TAB_ROWS + BLK_R - 1) // BLK_R
OUT_ROWS = RELAYOUT_GRID * BLK_R
BLK_SHIFT = BLK_R.bit_length() - 1
BLK_MASK = BLK_R - 1


def _relayout_body(u_ref, i_ref, uo_ref, io_ref):
    for x_ref, o_ref in ((u_ref, uo_ref), (i_ref, io_ref)):
        x = x_ref[...]
        xs = jnp.concatenate(
            [x[:, c * BLK_R:(c + 1) * BLK_R] for c in range(ROWS_PER_LINE)],
            axis=0)
        o_ref[...] = jnp.swapaxes(xs, 0, 1)


def _relayout(ut_t, it_t):
    spec_in = pl.BlockSpec((DIM, BLK_U), lambda i: (0, i))
    spec_out = pl.BlockSpec((BLK_R, 128), lambda i: (i, 0))
    return pl.pallas_call(
        _relayout_body,
        grid=(RELAYOUT_GRID,),
        in_specs=[spec_in, spec_in],
        out_specs=[spec_out, spec_out],
        out_shape=[jax.ShapeDtypeStruct((OUT_ROWS, 128), jnp.float32)] * 2,
    )(ut_t, it_t)


def _gmf_body(u_rm, i_rm, w_hbm, b_hbm, uidx_hbm, iidx_hbm, out_hbm,
              uidx_f, iidx_f, urid, irid, ubuf, ibuf, w_v, b_v, out_v, sem):
    wid = lax.axis_index("s") * NUM_CORES + lax.axis_index("c")
    base = wid * ROWS_PER_WORKER

    pltpu.sync_copy(uidx_hbm.at[pl.ds(base, ROWS_PER_WORKER)], uidx_f)
    pltpu.sync_copy(iidx_hbm.at[pl.ds(base, ROWS_PER_WORKER)], iidx_f)
    pltpu.sync_copy(w_hbm, w_v)
    pltpu.sync_copy(b_hbm, b_v)

    for j in range(ROWS_PER_WORKER // LANES):
        dst = (j * LANES) // CHUNK, pl.ds((j * LANES) % CHUNK, LANES)
        uv = uidx_f[pl.ds(j * LANES, LANES)]
        iv = iidx_f[pl.ds(j * LANES, LANES)]
        urid[dst] = ((uv >> (BLK_SHIFT + 2)) << BLK_SHIFT) + (uv & BLK_MASK)
        irid[dst] = ((iv >> (BLK_SHIFT + 2)) << BLK_SHIFT) + (iv & BLK_MASK)

    iota = lax.iota(jnp.int32, LANES)
    bias = b_v[...]
    w_lo = w_v[pl.ds(0, LANES)]
    w_hi = w_v[pl.ds(LANES, LANES)]

    for c in range(NCHUNKS):
        cu = pltpu.async_copy(u_rm.at[urid.at[c]], ubuf, sem)
        ci = pltpu.async_copy(i_rm.at[irid.at[c]], ibuf, sem)
        cu.wait()
        ci.wait()

        def group(g, carry, c=c):
            off = c * CHUNK + g * LANES
            uix = uidx_f[pl.ds(off, LANES)]
            iix = iidx_f[pl.ds(off, LANES)]
            rid = g * LANES + iota
            ul0 = ((uix >> BLK_SHIFT) & 3) * DIM
            il0 = ((iix >> BLK_SHIFT) & 3) * DIM
            acc = bias
            for d in range(DIM):
                ug = plsc.load_gather(ubuf, [rid, ul0 + d])
                vg = plsc.load_gather(ibuf, [rid, il0 + d])
                w_s = (w_lo if d < LANES else w_hi)[d % LANES]
                acc = acc + ug * vg * w_s
            out_v[pl.ds(off, LANES)] = 1.0 / (1.0 + jnp.exp(-acc))
            return carry

        lax.fori_loop(0, GROUPS, group, 0)

    pltpu.sync_copy(out_v, out_hbm.at[pl.ds(base, ROWS_PER_WORKER)])


@functools.partial(jax.jit, static_argnames=())
def _gmf(u_rm, i_rm, w_flat, b_vec, uidx, iidx):
    mesh = plsc.VectorSubcoreMesh(core_axis_name="c", subcore_axis_name="s")
    run = pl.kernel(
        _gmf_body,
        out_type=jax.ShapeDtypeStruct((BATCH,), jnp.float32),
        mesh=mesh,
        scratch_types=[
            pltpu.VMEM((ROWS_PER_WORKER,), jnp.int32),
            pltpu.VMEM((ROWS_PER_WORKER,), jnp.int32),
            pltpu.VMEM((NCHUNKS, CHUNK), jnp.int32),
            pltpu.VMEM((NCHUNKS, CHUNK), jnp.int32),
            pltpu.VMEM((CHUNK, 128), jnp.float32),
            pltpu.VMEM((CHUNK, 128), jnp.float32),
            pltpu.VMEM((DIM,), jnp.float32),
            pltpu.VMEM((LANES,), jnp.float32),
            pltpu.VMEM((ROWS_PER_WORKER,), jnp.float32),
            pltpu.SemaphoreType.DMA,
        ],
        compiler_params=pltpu.CompilerParams(
            needs_layout_passes=False, use_tc_tiling_on_sc=True),
    )
    return run(u_rm, i_rm, w_flat, b_vec, uidx, iidx)


def kernel(user_table, item_table, fc_w, fc_b, user_indices, item_indices):
    u_rm, i_rm = _relayout(user_table.T, item_table.T)
    w_flat = fc_w.reshape(DIM).astype(jnp.float32)
    b_vec = jnp.broadcast_to(fc_b.astype(jnp.float32), (LANES,))
    uidx = user_indices.astype(jnp.int32)
    iidx = item_indices.astype(jnp.int32)
    out = _gmf(u_rm, i_rm, w_flat, b_vec, uidx, iidx)
    return out.reshape(BATCH, 1)


# speedup vs baseline: 4.5193x; 1.0176x over previous
import functools

import jax
import jax.numpy as jnp
from jax import lax
from jax.experimental import pallas as pl
from jax.experimental.pallas import tpu as pltpu
from jax.experimental.pallas import tpu_sc as plsc

NUM_CORES = 2
NUM_SUBCORES = 16
NUM_WORKERS = NUM_CORES * NUM_SUBCORES
LANES = 16

BATCH = 16384
DIM = 32
NUM_ROWS = 1000000
ROWS_PER_LINE = 128 // DIM
TAB_ROWS = NUM_ROWS // ROWS_PER_LINE
ROWS_PER_WORKER = BATCH // NUM_WORKERS
CHUNK = 128
NCHUNKS = ROWS_PER_WORKER // CHUNK
GROUPS = CHUNK // LANES

BLK_R = 8192
BLK_U = BLK_R * ROWS_PER_LINE
RELAYOUT_GRID = (TAB_ROWS + BLK_R - 1) // BLK_R
OUT_ROWS = RELAYOUT_GRID * BLK_R
BLK_SHIFT = BLK_R.bit_length() - 1
BLK_MASK = BLK_R - 1


def _relayout_body(u_ref, i_ref, uo_ref, io_ref):
    for x_ref, o_ref in ((u_ref, uo_ref), (i_ref, io_ref)):
        x = x_ref[...]
        xs = jnp.concatenate(
            [x[:, c * BLK_R:(c + 1) * BLK_R] for c in range(ROWS_PER_LINE)],
            axis=0)
        o_ref[...] = jnp.swapaxes(xs, 0, 1)


def _relayout(ut_t, it_t):
    spec_in = pl.BlockSpec((DIM, BLK_U), lambda i: (0, i))
    spec_out = pl.BlockSpec((BLK_R, 128), lambda i: (i, 0))
    return pl.pallas_call(
        _relayout_body,
        grid=(RELAYOUT_GRID,),
        in_specs=[spec_in, spec_in],
        out_specs=[spec_out, spec_out],
        out_shape=[jax.ShapeDtypeStruct((OUT_ROWS, 128), jnp.float32)] * 2,
    )(ut_t, it_t)


def _gmf_body(u_rm, i_rm, w_hbm, b_hbm, uidx_hbm, iidx_hbm, out_hbm,
              uidx_f, iidx_f, urid, irid, ubuf, ibuf, w_v, b_v, out_v, sem):
    wid = lax.axis_index("s") * NUM_CORES + lax.axis_index("c")
    base = wid * ROWS_PER_WORKER

    pltpu.sync_copy(uidx_hbm.at[pl.ds(base, ROWS_PER_WORKER)], uidx_f)
    pltpu.sync_copy(iidx_hbm.at[pl.ds(base, ROWS_PER_WORKER)], iidx_f)
    pltpu.sync_copy(w_hbm, w_v)
    pltpu.sync_copy(b_hbm, b_v)

    for j in range(ROWS_PER_WORKER // LANES):
        dst = (j * LANES) // CHUNK, pl.ds((j * LANES) % CHUNK, LANES)
        uv = uidx_f[pl.ds(j * LANES, LANES)]
        iv = iidx_f[pl.ds(j * LANES, LANES)]
        urid[dst] = ((uv >> (BLK_SHIFT + 2)) << BLK_SHIFT) + (uv & BLK_MASK)
        irid[dst] = ((iv >> (BLK_SHIFT + 2)) << BLK_SHIFT) + (iv & BLK_MASK)

    iota = lax.iota(jnp.int32, LANES)
    bias = b_v[...]
    w_lo = w_v[pl.ds(0, LANES)]
    w_hi = w_v[pl.ds(LANES, LANES)]

    def fire(c):
        par = c % 2
        s = sem.at[par]
        return (pltpu.async_copy(u_rm.at[urid.at[c]], ubuf.at[par], s),
                pltpu.async_copy(i_rm.at[irid.at[c]], ibuf.at[par], s))

    pending = fire(0)
    for c in range(NCHUNKS):
        nxt = fire(c + 1) if c + 1 < NCHUNKS else None
        for cp in pending:
            cp.wait()
        ub = ubuf.at[c % 2]
        ib = ibuf.at[c % 2]

        def group(g, carry, c=c, ub=ub, ib=ib):
            off = c * CHUNK + g * LANES
            uix = uidx_f[pl.ds(off, LANES)]
            iix = iidx_f[pl.ds(off, LANES)]
            rid = g * LANES + iota
            ul0 = ((uix >> BLK_SHIFT) & 3) * DIM
            il0 = ((iix >> BLK_SHIFT) & 3) * DIM
            acc = bias
            for d in range(DIM):
                ug = plsc.load_gather(ub, [rid, ul0 + d])
                vg = plsc.load_gather(ib, [rid, il0 + d])
                w_s = (w_lo if d < LANES else w_hi)[d % LANES]
                acc = acc + ug * vg * w_s
            out_v[pl.ds(off, LANES)] = 1.0 / (1.0 + jnp.exp(-acc))
            return carry

        lax.fori_loop(0, GROUPS, group, 0)
        pending = nxt

    pltpu.sync_copy(out_v, out_hbm.at[pl.ds(base, ROWS_PER_WORKER)])


@functools.partial(jax.jit, static_argnames=())
def _gmf(u_rm, i_rm, w_flat, b_vec, uidx, iidx):
    mesh = plsc.VectorSubcoreMesh(core_axis_name="c", subcore_axis_name="s")
    run = pl.kernel(
        _gmf_body,
        out_type=jax.ShapeDtypeStruct((BATCH,), jnp.float32),
        mesh=mesh,
        scratch_types=[
            pltpu.VMEM((ROWS_PER_WORKER,), jnp.int32),
            pltpu.VMEM((ROWS_PER_WORKER,), jnp.int32),
            pltpu.VMEM((NCHUNKS, CHUNK), jnp.int32),
            pltpu.VMEM((NCHUNKS, CHUNK), jnp.int32),
            pltpu.VMEM((2, CHUNK, 128), jnp.float32),
            pltpu.VMEM((2, CHUNK, 128), jnp.float32),
            pltpu.VMEM((DIM,), jnp.float32),
            pltpu.VMEM((LANES,), jnp.float32),
            pltpu.VMEM((ROWS_PER_WORKER,), jnp.float32),
            pltpu.SemaphoreType.DMA((2,)),
        ],
        compiler_params=pltpu.CompilerParams(
            needs_layout_passes=False, use_tc_tiling_on_sc=True),
    )
    return run(u_rm, i_rm, w_flat, b_vec, uidx, iidx)


def kernel(user_table, item_table, fc_w, fc_b, user_indices, item_indices):
    u_rm, i_rm = _relayout(user_table.T, item_table.T)
    w_flat = fc_w.reshape(DIM).astype(jnp.float32)
    b_vec = jnp.broadcast_to(fc_b.astype(jnp.float32), (LANES,))
    uidx = user_indices.astype(jnp.int32)
    iidx = item_indices.astype(jnp.int32)
    out = _gmf(u_rm, i_rm, w_flat, b_vec, uidx, iidx)
    return out.reshape(BATCH, 1)


# speedup vs baseline: 4.6040x; 1.0188x over previous
import functools

import jax
import jax.numpy as jnp
from jax import lax
from jax.experimental import pallas as pl
from jax.experimental.pallas import tpu as pltpu
from jax.experimental.pallas import tpu_sc as plsc

NUM_CORES = 2
NUM_SUBCORES = 16
NUM_WORKERS = NUM_CORES * NUM_SUBCORES
LANES = 16

BATCH = 16384
DIM = 32
NUM_ROWS = 1000000
ROWS_PER_LINE = 128 // DIM
TAB_ROWS = NUM_ROWS // ROWS_PER_LINE
ROWS_PER_WORKER = BATCH // NUM_WORKERS
CHUNK = 128
NCHUNKS = ROWS_PER_WORKER // CHUNK
GROUPS = CHUNK // LANES

BLK_R = 8192
BLK_U = BLK_R * ROWS_PER_LINE
RELAYOUT_GRID = (TAB_ROWS + BLK_R - 1) // BLK_R
OUT_ROWS = RELAYOUT_GRID * BLK_R
BLK_SHIFT = BLK_R.bit_length() - 1
BLK_MASK = BLK_R - 1


def _relayout_body(u_ref, i_ref, w_ref, uo_ref, io_ref):
    for x_ref, o_ref, scale in ((u_ref, uo_ref, None), (i_ref, io_ref, w_ref)):
        x = x_ref[...]
        if scale is not None:
            x = x * scale[...][:, None]
        xs = jnp.concatenate(
            [x[:, c * BLK_R:(c + 1) * BLK_R] for c in range(ROWS_PER_LINE)],
            axis=0)
        o_ref[...] = jnp.swapaxes(xs, 0, 1)


def _relayout(ut_t, it_t, w_flat):
    spec_in = pl.BlockSpec((DIM, BLK_U), lambda i: (0, i))
    spec_out = pl.BlockSpec((BLK_R, 128), lambda i: (i, 0))
    return pl.pallas_call(
        _relayout_body,
        grid=(RELAYOUT_GRID,),
        in_specs=[spec_in, spec_in, pl.BlockSpec((DIM,), lambda i: (0,))],
        out_specs=[spec_out, spec_out],
        out_shape=[jax.ShapeDtypeStruct((OUT_ROWS, 128), jnp.float32)] * 2,
    )(ut_t, it_t, w_flat)


def _gmf_body(u_rm, i_rm, b_hbm, uidx_hbm, iidx_hbm, out_hbm,
              uidx_f, iidx_f, urid, irid, ubuf, ibuf, b_v, out_v, sem):
    wid = lax.axis_index("s") * NUM_CORES + lax.axis_index("c")
    base = wid * ROWS_PER_WORKER

    pltpu.sync_copy(uidx_hbm.at[pl.ds(base, ROWS_PER_WORKER)], uidx_f)
    pltpu.sync_copy(iidx_hbm.at[pl.ds(base, ROWS_PER_WORKER)], iidx_f)
    pltpu.sync_copy(b_hbm, b_v)

    for j in range(ROWS_PER_WORKER // LANES):
        dst = (j * LANES) // CHUNK, pl.ds((j * LANES) % CHUNK, LANES)
        uv = uidx_f[pl.ds(j * LANES, LANES)]
        iv = iidx_f[pl.ds(j * LANES, LANES)]
        urid[dst] = ((uv >> (BLK_SHIFT + 2)) << BLK_SHIFT) + (uv & BLK_MASK)
        irid[dst] = ((iv >> (BLK_SHIFT + 2)) << BLK_SHIFT) + (iv & BLK_MASK)

    iota = lax.iota(jnp.int32, LANES)
    bias = b_v[...]

    def fire(c):
        par = c % 2
        s = sem.at[par]
        return (pltpu.async_copy(u_rm.at[urid.at[c]], ubuf.at[par], s),
                pltpu.async_copy(i_rm.at[irid.at[c]], ibuf.at[par], s))

    pending = fire(0)
    for c in range(NCHUNKS):
        nxt = fire(c + 1) if c + 1 < NCHUNKS else None
        for cp in pending:
            cp.wait()
        ub = ubuf.at[c % 2]
        ib = ibuf.at[c % 2]

        def group(g, carry, c=c, ub=ub, ib=ib):
            off = c * CHUNK + g * LANES
            uix = uidx_f[pl.ds(off, LANES)]
            iix = iidx_f[pl.ds(off, LANES)]
            rid = g * LANES + iota
            ul0 = ((uix >> BLK_SHIFT) & 3) * DIM
            il0 = ((iix >> BLK_SHIFT) & 3) * DIM
            acc0 = bias
            acc1 = jnp.zeros((LANES,), jnp.float32)
            for d in range(0, DIM, 2):
                ug0 = plsc.load_gather(ub, [rid, ul0 + d])
                vg0 = plsc.load_gather(ib, [rid, il0 + d])
                ug1 = plsc.load_gather(ub, [rid, ul0 + (d + 1)])
                vg1 = plsc.load_gather(ib, [rid, il0 + (d + 1)])
                acc0 = acc0 + ug0 * vg0
                acc1 = acc1 + ug1 * vg1
            acc = acc0 + acc1
            out_v[pl.ds(off, LANES)] = 1.0 / (1.0 + jnp.exp(-acc))
            return carry

        lax.fori_loop(0, GROUPS, group, 0)
        pending = nxt

    pltpu.sync_copy(out_v, out_hbm.at[pl.ds(base, ROWS_PER_WORKER)])


@functools.partial(jax.jit, static_argnames=())
def _gmf(u_rm, i_rm, b_vec, uidx, iidx):
    mesh = plsc.VectorSubcoreMesh(core_axis_name="c", subcore_axis_name="s")
    run = pl.kernel(
        _gmf_body,
        out_type=jax.ShapeDtypeStruct((BATCH,), jnp.float32),
        mesh=mesh,
        scratch_types=[
            pltpu.VMEM((ROWS_PER_WORKER,), jnp.int32),
            pltpu.VMEM((ROWS_PER_WORKER,), jnp.int32),
            pltpu.VMEM((NCHUNKS, CHUNK), jnp.int32),
            pltpu.VMEM((NCHUNKS, CHUNK), jnp.int32),
            pltpu.VMEM((2, CHUNK, 128), jnp.float32),
            pltpu.VMEM((2, CHUNK, 128), jnp.float32),
            pltpu.VMEM((LANES,), jnp.float32),
            pltpu.VMEM((ROWS_PER_WORKER,), jnp.float32),
            pltpu.SemaphoreType.DMA((2,)),
        ],
        compiler_params=pltpu.CompilerParams(
            needs_layout_passes=False, use_tc_tiling_on_sc=True),
    )
    return run(u_rm, i_rm, b_vec, uidx, iidx)


def kernel(user_table, item_table, fc_w, fc_b, user_indices, item_indices):
    w_flat = fc_w.reshape(DIM).astype(jnp.float32)
    u_rm, i_rm = _relayout(user_table.T, item_table.T, w_flat)
    b_vec = jnp.broadcast_to(fc_b.astype(jnp.float32), (LANES,))
    uidx = user_indices.astype(jnp.int32)
    iidx = item_indices.astype(jnp.int32)
    out = _gmf(u_rm, i_rm, b_vec, uidx, iidx)
    return out.reshape(BATCH, 1)
